# Initial kernel scaffold; baseline (speedup 1.0000x reference)
import jax
import jax.numpy as jnp
from jax.experimental import pallas as pl


def kernel(h, c, edge_index, Wq, Wk, Wv, Wp1, bp1, Wp2, bp2, We1, be1, We2, be2, Wo, bo):
    raise NotImplementedError("write your pallas kernel here")


# speedup vs baseline: 15.3487x; 15.3487x over previous
import functools

import jax
import jax.numpy as jnp
from jax import lax
from jax.experimental import pallas as pl
from jax.experimental.pallas import tpu as pltpu
from jax.experimental.pallas import tpu_sc as plsc

N = 10000
E = 320000
D = 128
H = 8
DH = 16

NC = 2
NS = 16
NW = NC * NS
EPW = E // NW
EB = 80
NB = EPW // EB
GPB = EB // 16
ACCW = 144
CH = 80
NCHK = N // CH


def _s1_body(h_ref, c_ref, wp1_ref, bp1_ref, wsrc_ref, wp2_ref, bsrc_ref,
             wqq_ref, bqw_ref, oaw_ref, ovp_ref, oqw_ref):
    hb = h_ref[...]
    p2 = jnp.maximum(c_ref[...] @ wp1_ref[...] + bp1_ref[...], 0.0)
    big = hb @ wsrc_ref[...] + p2 @ wp2_ref[...] + bsrc_ref[...]
    oaw_ref[...] = big[:, :D]
    ovp_ref[...] = big[:, D:]
    oqw_ref[...] = hb @ wqq_ref[...] + bqw_ref[...]


_R1 = 1000
_s1_call = pl.pallas_call(
    _s1_body,
    grid=(N // _R1,),
    in_specs=[
        pl.BlockSpec((_R1, D), lambda i: (i, 0)),
        pl.BlockSpec((_R1, 8), lambda i: (i, 0)),
        pl.BlockSpec((8, 8), lambda i: (0, 0)),
        pl.BlockSpec((1, 8), lambda i: (0, 0)),
        pl.BlockSpec((D, 2 * D), lambda i: (0, 0)),
        pl.BlockSpec((8, 2 * D), lambda i: (0, 0)),
        pl.BlockSpec((1, 2 * D), lambda i: (0, 0)),
        pl.BlockSpec((D, D), lambda i: (0, 0)),
        pl.BlockSpec((1, D), lambda i: (0, 0)),
    ],
    out_specs=[
        pl.BlockSpec((_R1, D), lambda i: (i, 0)),
        pl.BlockSpec((_R1, D), lambda i: (i, 0)),
        pl.BlockSpec((_R1, D), lambda i: (i, 0)),
    ],
    out_shape=[
        jax.ShapeDtypeStruct((N, D), jnp.float32),
        jax.ShapeDtypeStruct((N, D), jnp.float32),
        jax.ShapeDtypeStruct((N, D), jnp.float32),
    ],
)


def _edge_body(src_hbm, dst_hbm, aw_hbm, vp_hbm, qw_hbm, cons_hbm, out0, out1,
               srcb, dstb, rb, qwrows, msg, cons, acc, sem1, sem2):
    cid = lax.axis_index("c")
    sid = lax.axis_index("s")
    wid = cid * NS + sid

    pltpu.sync_copy(cons_hbm, cons)

    zv = jnp.zeros((16,), jnp.float32)

    def mrow(r, _):
        for c9 in range(ACCW // 16):
            msg[r, pl.ds(c9 * 16, 16)] = zv
        return 0

    lax.fori_loop(0, EB, mrow, 0)

    nchk_t = 7 + jnp.int32(sid < NCHK - 7 * NS)

    def zchunk(i, _):
        pltpu.sync_copy(msg, acc.at[pl.ds((sid + i * NS) * CH, CH)])
        return 0

    lax.fori_loop(0, nchk_t, zchunk, 0)
    plsc.subcore_barrier()

    iota16 = lax.broadcasted_iota(jnp.int32, (16,), 0)
    ebase = wid * EPW

    def batch(b, _):
        off = ebase + b * EB
        pltpu.sync_copy(src_hbm.at[pl.ds(off, EB)], srcb)
        pltpu.sync_copy(dst_hbm.at[pl.ds(off, EB)], dstb)
        cp1 = pltpu.async_copy(aw_hbm.at[srcb], rb, sem1)
        cp2 = pltpu.async_copy(qw_hbm.at[dstb], qwrows, sem2)
        cp1.wait()
        cp2.wait()

        w2v = cons[pl.ds(0, 16)]
        b2v = cons[pl.ds(16, 16)]

        def sgroup(g, _):
            rows = g * 16 + iota16
            accs = [jnp.zeros((16,), jnp.float32)] * H
            for d in range(DH):
                w2 = w2v[d]
                for h in range(H):
                    colv = jnp.full((16,), h * DH + d, jnp.int32)
                    aw = plsc.load_gather(rb, [rows, colv])
                    qv = plsc.load_gather(qwrows, [rows, colv])
                    t = jnp.maximum(aw - qv, 0.0)
                    accs[h] = accs[h] + t * w2
            for h in range(H):
                sc = jnp.exp(jnp.minimum(jnp.maximum(accs[h] + b2v,
                                                     -5.0), 5.0))
                plsc.store_scatter(
                    msg, [rows, jnp.full((16,), D + h, jnp.int32)], sc)
            return 0

        lax.fori_loop(0, GPB, sgroup, 0)

        cp3 = pltpu.async_copy(vp_hbm.at[srcb], rb, sem1)
        cp3.wait()

        def mgroup(g, _):
            rows = g * 16 + iota16
            scores = [plsc.load_gather(
                msg, [rows, jnp.full((16,), D + h, jnp.int32)])
                for h in range(H)]
            for d in range(DH):
                for h in range(H):
                    colv = jnp.full((16,), h * DH + d, jnp.int32)
                    vp = plsc.load_gather(rb, [rows, colv])
                    plsc.store_scatter(msg, [rows, colv], vp * scores[h])
            return 0

        lax.fori_loop(0, GPB, mgroup, 0)
        pltpu.sync_copy(msg, acc.at[dstb], add=True)
        return 0

    lax.fori_loop(0, NB, batch, 0)
    plsc.subcore_barrier()

    def dchunk(i, _):
        r0 = (sid + i * NS) * CH

        @pl.when(cid == 0)
        def _():
            pltpu.sync_copy(acc.at[pl.ds(r0, CH)], out0.at[pl.ds(r0, CH)])

        @pl.when(cid == 1)
        def _():
            pltpu.sync_copy(acc.at[pl.ds(r0, CH)], out1.at[pl.ds(r0, CH)])

        return 0

    lax.fori_loop(0, nchk_t, dchunk, 0)


_edge_call = functools.partial(
    pl.kernel,
    out_type=(
        jax.ShapeDtypeStruct((N, ACCW), jnp.float32),
        jax.ShapeDtypeStruct((N, ACCW), jnp.float32),
    ),
    mesh=plsc.VectorSubcoreMesh(core_axis_name="c", subcore_axis_name="s"),
    compiler_params=pltpu.CompilerParams(
        use_tc_tiling_on_sc=False, needs_layout_passes=False),
    scratch_types=[
        pltpu.VMEM((EB,), jnp.int32),
        pltpu.VMEM((EB,), jnp.int32),
        pltpu.VMEM((EB, D), jnp.float32),
        pltpu.VMEM((EB, D), jnp.float32),
        pltpu.VMEM((EB, ACCW), jnp.float32),
        pltpu.VMEM((32,), jnp.float32),
        pltpu.VMEM_SHARED((N, ACCW), jnp.float32),
        pltpu.SemaphoreType.DMA,
        pltpu.SemaphoreType.DMA,
    ],
)(_edge_body)


def _s3_body(a0_ref, a1_ref, r8_ref, wo_ref, bo_ref, o_ref):
    wv = a0_ref[:, :D] + a1_ref[:, :D]
    z = a0_ref[:, D:D + H] + a1_ref[:, D:D + H]
    zr = (1.0 / z) @ r8_ref[...]
    o_ref[...] = (wv * zr) @ wo_ref[...] + bo_ref[...]


_s3_call = pl.pallas_call(
    _s3_body,
    grid=(N // _R1,),
    in_specs=[
        pl.BlockSpec((_R1, ACCW), lambda i: (i, 0)),
        pl.BlockSpec((_R1, ACCW), lambda i: (i, 0)),
        pl.BlockSpec((H, D), lambda i: (0, 0)),
        pl.BlockSpec((D, D), lambda i: (0, 0)),
        pl.BlockSpec((1, D), lambda i: (0, 0)),
    ],
    out_specs=pl.BlockSpec((_R1, D), lambda i: (i, 0)),
    out_shape=jax.ShapeDtypeStruct((N, D), jnp.float32),
)


def kernel(h, c, edge_index, Wq, Wk, Wv, Wp1, bp1, Wp2, bp2,
           We1, be1, We2, be2, Wo, bo):
    f32 = jnp.float32
    bd = jnp.kron(jnp.eye(H, dtype=f32), We1)
    wsrc = jnp.concatenate([Wk @ bd, Wv], axis=1)
    wp2src = jnp.concatenate([Wp2 @ bd, Wp2], axis=1)
    wp2src = jnp.pad(wp2src, ((0, 5), (0, 0)))
    bsrc = jnp.concatenate([bp2 @ bd, bp2]).reshape(1, 2 * D)
    wqq = Wq @ bd
    bqw = (-jnp.tile(be1, H)).reshape(1, D)
    wp1p = jnp.pad(Wp1, ((0, 5), (0, 5)))
    bp1p = jnp.pad(bp1, (0, 5)).reshape(1, 8)
    cpad = jnp.pad(c, ((0, 0), (0, 5)))

    awn, vpn, qwn = _s1_call(h, cpad, wp1p, bp1p, wsrc, wp2src, bsrc,
                             wqq, bqw)

    cons = jnp.concatenate([We2[:, 0] * 0.25,
                            jnp.full((16,), be2[0] * 0.25, f32)])
    src = edge_index[0]
    dst = edge_index[1]
    a0, a1 = _edge_call(src, dst, awn, vpn, qwn, cons)

    r8 = jnp.kron(jnp.eye(H, dtype=f32), jnp.ones((1, DH), f32))
    return _s3_call(a0, a1, r8, Wo, bo.reshape(1, D))


# speedup vs baseline: 18.6256x; 1.2135x over previous
import functools

import jax
import jax.numpy as jnp
from jax import lax
from jax.experimental import pallas as pl
from jax.experimental.pallas import tpu as pltpu
from jax.experimental.pallas import tpu_sc as plsc

N = 10000
E = 320000
D = 128
H = 8
DH = 16

NC = 2
NS = 16
NW = NC * NS
EPW = E // NW
EB = 80
NB = EPW // EB
GPB = EB // 16
ACCW = 144
CH = 80
NCHK = N // CH


def _s1_body(h_ref, c_ref, wp1_ref, bp1_ref, wsrc_ref, wp2_ref, bsrc_ref,
             wqq_ref, bqw_ref, oaw_ref, ovp_ref, oqw_ref):
    hb = h_ref[...]
    p2 = jnp.maximum(c_ref[...] @ wp1_ref[...] + bp1_ref[...], 0.0)
    big = hb @ wsrc_ref[...] + p2 @ wp2_ref[...] + bsrc_ref[...]
    oaw_ref[...] = big[:, :D]
    ovp_ref[...] = jnp.concatenate(
        [big[:, D:], jnp.zeros((big.shape[0], ACCW - D), jnp.float32)],
        axis=1)
    oqw_ref[...] = hb @ wqq_ref[...] + bqw_ref[...]


_R1 = 1000
_s1_call = pl.pallas_call(
    _s1_body,
    grid=(N // _R1,),
    in_specs=[
        pl.BlockSpec((_R1, D), lambda i: (i, 0)),
        pl.BlockSpec((_R1, 8), lambda i: (i, 0)),
        pl.BlockSpec((8, 8), lambda i: (0, 0)),
        pl.BlockSpec((1, 8), lambda i: (0, 0)),
        pl.BlockSpec((D, 2 * D), lambda i: (0, 0)),
        pl.BlockSpec((8, 2 * D), lambda i: (0, 0)),
        pl.BlockSpec((1, 2 * D), lambda i: (0, 0)),
        pl.BlockSpec((D, D), lambda i: (0, 0)),
        pl.BlockSpec((1, D), lambda i: (0, 0)),
    ],
    out_specs=[
        pl.BlockSpec((_R1, D), lambda i: (i, 0)),
        pl.BlockSpec((_R1, ACCW), lambda i: (i, 0)),
        pl.BlockSpec((_R1, D), lambda i: (i, 0)),
    ],
    out_shape=[
        jax.ShapeDtypeStruct((N, D), jnp.float32),
        jax.ShapeDtypeStruct((N, ACCW), jnp.float32),
        jax.ShapeDtypeStruct((N, D), jnp.float32),
    ],
)


def _edge_body(src_hbm, dst_hbm, aw_hbm, vp_hbm, qw_hbm, cons_hbm, out0, out1,
               idxb, rb, qwrows, msg, cons, acc, semi, semg):
    cid = lax.axis_index("c")
    sid = lax.axis_index("s")
    wid = cid * NS + sid

    pltpu.sync_copy(cons_hbm, cons)

    zv = jnp.zeros((16,), jnp.float32)

    def mrow(r, _):
        for c9 in range(ACCW // 16):
            msg[r, pl.ds(c9 * 16, 16)] = zv
        return 0

    lax.fori_loop(0, EB, mrow, 0)

    nchk_t = 7 + jnp.int32(sid < NCHK - 7 * NS)

    def zchunk(i, _):
        pltpu.sync_copy(msg, acc.at[pl.ds((sid + i * NS) * CH, CH)])
        return 0

    lax.fori_loop(0, nchk_t, zchunk, 0)
    plsc.subcore_barrier()

    iota16 = lax.broadcasted_iota(jnp.int32, (16,), 0)
    ebase = wid * EPW

    pltpu.async_copy(src_hbm.at[pl.ds(ebase, EB)], idxb.at[0], semi)
    pltpu.async_copy(dst_hbm.at[pl.ds(ebase, EB)], idxb.at[2], semi)

    w2v = cons[pl.ds(0, 16)]
    b2v = cons[pl.ds(16, 16)]

    def batch(b, _):
        k = b % 2
        kn = (b + 1) % 2
        pltpu.make_async_copy(
            src_hbm.at[pl.ds(0, EB)], idxb.at[k], semi).wait()
        pltpu.make_async_copy(
            src_hbm.at[pl.ds(0, EB)], idxb.at[2 + k], semi).wait()

        g1 = pltpu.async_copy(aw_hbm.at[idxb.at[k]], rb, semg)
        g2 = pltpu.async_copy(qw_hbm.at[idxb.at[2 + k]], qwrows, semg)
        g3 = pltpu.async_copy(vp_hbm.at[idxb.at[k]], msg, semg)

        @pl.when(b + 1 < NB)
        def _():
            off_n = ebase + (b + 1) * EB
            pltpu.async_copy(src_hbm.at[pl.ds(off_n, EB)], idxb.at[kn], semi)
            pltpu.async_copy(
                dst_hbm.at[pl.ds(off_n, EB)], idxb.at[2 + kn], semi)

        g1.wait()
        g2.wait()
        g3.wait()

        def group(g, _):
            rows = g * 16 + iota16
            accs = [jnp.zeros((16,), jnp.float32)] * H
            for d in range(DH):
                w2 = w2v[d]
                for h in range(H):
                    colv = jnp.full((16,), h * DH + d, jnp.int32)
                    aw = plsc.load_gather(rb, [rows, colv])
                    qv = plsc.load_gather(qwrows, [rows, colv])
                    t = jnp.maximum(aw - qv, 0.0)
                    accs[h] = accs[h] + t * w2
            scores = []
            for h in range(H):
                sc = jnp.exp(jnp.minimum(jnp.maximum(accs[h] + b2v,
                                                     -5.0), 5.0))
                scores.append(sc)
                plsc.store_scatter(
                    msg, [rows, jnp.full((16,), D + h, jnp.int32)], sc)
            for d in range(DH):
                for h in range(H):
                    colv = jnp.full((16,), h * DH + d, jnp.int32)
                    vp = plsc.load_gather(msg, [rows, colv])
                    plsc.store_scatter(msg, [rows, colv], vp * scores[h])
            return 0

        lax.fori_loop(0, GPB, group, 0)
        pltpu.sync_copy(msg, acc.at[idxb.at[2 + k]], add=True)
        return 0

    lax.fori_loop(0, NB, batch, 0)
    plsc.subcore_barrier()

    def dchunk(i, _):
        r0 = (sid + i * NS) * CH

        @pl.when(cid == 0)
        def _():
            pltpu.sync_copy(acc.at[pl.ds(r0, CH)], out0.at[pl.ds(r0, CH)])

        @pl.when(cid == 1)
        def _():
            pltpu.sync_copy(acc.at[pl.ds(r0, CH)], out1.at[pl.ds(r0, CH)])

        return 0

    lax.fori_loop(0, nchk_t, dchunk, 0)


_edge_call = functools.partial(
    pl.kernel,
    out_type=(
        jax.ShapeDtypeStruct((N, ACCW), jnp.float32),
        jax.ShapeDtypeStruct((N, ACCW), jnp.float32),
    ),
    mesh=plsc.VectorSubcoreMesh(core_axis_name="c", subcore_axis_name="s"),
    compiler_params=pltpu.CompilerParams(
        use_tc_tiling_on_sc=False, needs_layout_passes=False),
    scratch_types=[
        pltpu.VMEM((4, EB), jnp.int32),
        pltpu.VMEM((EB, D), jnp.float32),
        pltpu.VMEM((EB, D), jnp.float32),
        pltpu.VMEM((EB, ACCW), jnp.float32),
        pltpu.VMEM((32,), jnp.float32),
        pltpu.VMEM_SHARED((N, ACCW), jnp.float32),
        pltpu.SemaphoreType.DMA,
        pltpu.SemaphoreType.DMA,
    ],
)(_edge_body)


def _s3_body(a0_ref, a1_ref, r8_ref, wo_ref, bo_ref, o_ref):
    wv = a0_ref[:, :D] + a1_ref[:, :D]
    z = a0_ref[:, D:D + H] + a1_ref[:, D:D + H]
    zr = (1.0 / z) @ r8_ref[...]
    o_ref[...] = (wv * zr) @ wo_ref[...] + bo_ref[...]


_s3_call = pl.pallas_call(
    _s3_body,
    grid=(N // _R1,),
    in_specs=[
        pl.BlockSpec((_R1, ACCW), lambda i: (i, 0)),
        pl.BlockSpec((_R1, ACCW), lambda i: (i, 0)),
        pl.BlockSpec((H, D), lambda i: (0, 0)),
        pl.BlockSpec((D, D), lambda i: (0, 0)),
        pl.BlockSpec((1, D), lambda i: (0, 0)),
    ],
    out_specs=pl.BlockSpec((_R1, D), lambda i: (i, 0)),
    out_shape=jax.ShapeDtypeStruct((N, D), jnp.float32),
)


def kernel(h, c, edge_index, Wq, Wk, Wv, Wp1, bp1, Wp2, bp2,
           We1, be1, We2, be2, Wo, bo):
    f32 = jnp.float32
    bd = jnp.kron(jnp.eye(H, dtype=f32), We1)
    wsrc = jnp.concatenate([Wk @ bd, Wv], axis=1)
    wp2src = jnp.concatenate([Wp2 @ bd, Wp2], axis=1)
    wp2src = jnp.pad(wp2src, ((0, 5), (0, 0)))
    bsrc = jnp.concatenate([bp2 @ bd, bp2]).reshape(1, 2 * D)
    wqq = Wq @ bd
    bqw = (-jnp.tile(be1, H)).reshape(1, D)
    wp1p = jnp.pad(Wp1, ((0, 5), (0, 5)))
    bp1p = jnp.pad(bp1, (0, 5)).reshape(1, 8)
    cpad = jnp.pad(c, ((0, 0), (0, 5)))

    awn, vpn, qwn = _s1_call(h, cpad, wp1p, bp1p, wsrc, wp2src, bsrc,
                             wqq, bqw)

    cons = jnp.concatenate([We2[:, 0] * 0.25,
                            jnp.full((16,), be2[0] * 0.25, f32)])
    src = edge_index[0]
    dst = edge_index[1]
    a0, a1 = _edge_call(src, dst, awn, vpn, qwn, cons)

    r8 = jnp.kron(jnp.eye(H, dtype=f32), jnp.ones((1, DH), f32))
    return _s3_call(a0, a1, r8, Wo, bo.reshape(1, D))


# speedup vs baseline: 20.0148x; 1.0746x over previous
import functools

import jax
import jax.numpy as jnp
from jax import lax
from jax.experimental import pallas as pl
from jax.experimental.pallas import tpu as pltpu
from jax.experimental.pallas import tpu_sc as plsc

N = 10000
E = 320000
D = 128
H = 8
DH = 16

NC = 2
NS = 16
NW = NC * NS
EPW = E // NW
EB = 48
NB = -(-EPW // EB)
LPW = NB * EB
GPB = EB // 16
ACCW = 144
N2 = N + 8
CH = 24
NCHK = N2 // CH


def _s1_body(h_ref, c_ref, wp1_ref, bp1_ref, wsrc_ref, wp2_ref, bsrc_ref,
             wqq_ref, bqw_ref, oaw_ref, ovp_ref, oqw_ref):
    hb = h_ref[...]
    p2 = jnp.maximum(c_ref[...] @ wp1_ref[...] + bp1_ref[...], 0.0)
    big = hb @ wsrc_ref[...] + p2 @ wp2_ref[...] + bsrc_ref[...]
    oaw_ref[...] = big[:, :D]
    ovp_ref[...] = jnp.concatenate(
        [big[:, D:], jnp.zeros((big.shape[0], ACCW - D), jnp.float32)],
        axis=1)
    oqw_ref[...] = hb @ wqq_ref[...] + bqw_ref[...]


_R1 = 1000
_s1_call = pl.pallas_call(
    _s1_body,
    grid=(N // _R1,),
    in_specs=[
        pl.BlockSpec((_R1, D), lambda i: (i, 0)),
        pl.BlockSpec((_R1, 8), lambda i: (i, 0)),
        pl.BlockSpec((8, 8), lambda i: (0, 0)),
        pl.BlockSpec((1, 8), lambda i: (0, 0)),
        pl.BlockSpec((D, 2 * D), lambda i: (0, 0)),
        pl.BlockSpec((8, 2 * D), lambda i: (0, 0)),
        pl.BlockSpec((1, 2 * D), lambda i: (0, 0)),
        pl.BlockSpec((D, D), lambda i: (0, 0)),
        pl.BlockSpec((1, D), lambda i: (0, 0)),
    ],
    out_specs=[
        pl.BlockSpec((_R1, D), lambda i: (i, 0)),
        pl.BlockSpec((_R1, ACCW), lambda i: (i, 0)),
        pl.BlockSpec((_R1, D), lambda i: (i, 0)),
    ],
    out_shape=[
        jax.ShapeDtypeStruct((N, D), jnp.float32),
        jax.ShapeDtypeStruct((N, ACCW), jnp.float32),
        jax.ShapeDtypeStruct((N, D), jnp.float32),
    ],
)


def _edge_body(src_hbm, dst_hbm, aw_hbm, vp_hbm, qw_hbm, cons_hbm, out0, out1,
               idxb, rb0, rb1, qw0, qw1, msg0, msg1, cons, acc,
               semi, semg0, semg1, sems0, sems1):
    cid = lax.axis_index("c")
    sid = lax.axis_index("s")
    wid = cid * NS + sid
    rbs, qws, msgs = (rb0, rb1), (qw0, qw1), (msg0, msg1)
    semg, sems = (semg0, semg1), (sems0, sems1)

    pltpu.sync_copy(cons_hbm, cons)

    zv = jnp.zeros((16,), jnp.float32)

    def mrow(r, _):
        for c9 in range(ACCW // 16):
            msg0[r, pl.ds(c9 * 16, 16)] = zv
        return 0

    lax.fori_loop(0, CH, mrow, 0)

    base_chunks = NCHK // NS
    nchk_t = base_chunks + jnp.int32(sid < NCHK - base_chunks * NS)

    def zchunk(i, _):
        pltpu.sync_copy(msg0.at[pl.ds(0, CH)],
                        acc.at[pl.ds((sid + i * NS) * CH, CH)])
        return 0

    lax.fori_loop(0, nchk_t, zchunk, 0)
    plsc.subcore_barrier()

    iota16 = lax.broadcasted_iota(jnp.int32, (16,), 0)

    w2v = cons[pl.ds(0, 16)]
    b2v = cons[pl.ds(16, 16)]

    def issue_idx(b):
        r = b % 4
        pltpu.async_copy(src_hbm.at[wid, pl.ds(b * EB, EB)],
                         idxb.at[r], semi)
        pltpu.async_copy(dst_hbm.at[wid, pl.ds(b * EB, EB)],
                         idxb.at[4 + r], semi)

    def drain_idx(b):
        r = b % 4
        pltpu.make_async_copy(
            src_hbm.at[0, pl.ds(0, EB)], idxb.at[r], semi).wait()
        pltpu.make_async_copy(
            src_hbm.at[0, pl.ds(0, EB)], idxb.at[4 + r], semi).wait()

    def issue_gathers(b, k):
        r = b % 4
        pltpu.async_copy(aw_hbm.at[idxb.at[r]], rbs[k], semg[k])
        pltpu.async_copy(qw_hbm.at[idxb.at[4 + r]], qws[k], semg[k])
        pltpu.async_copy(vp_hbm.at[idxb.at[r]], msgs[k], semg[k])

    def drain_gathers(k):
        pltpu.make_async_copy(aw_hbm.at[idxb.at[0]], rbs[k],
                              semg[k]).wait()
        pltpu.make_async_copy(qw_hbm.at[idxb.at[4]], qws[k],
                              semg[k]).wait()
        pltpu.make_async_copy(vp_hbm.at[idxb.at[0]], msgs[k],
                              semg[k]).wait()

    def wait_scatter(k):
        pltpu.make_async_copy(msgs[k], acc.at[idxb.at[4]],
                              sems[k]).wait()

    def compute_batch(b, k):
        r = b % 4
        rb, qwr, msg = rbs[k], qws[k], msgs[k]

        def group(g, _):
            rows = g * 16 + iota16
            accs = [jnp.zeros((16,), jnp.float32)] * H
            for d in range(DH):
                w2 = w2v[d]
                for h in range(H):
                    colv = jnp.full((16,), h * DH + d, jnp.int32)
                    aw = plsc.load_gather(rb, [rows, colv])
                    qv = plsc.load_gather(qwr, [rows, colv])
                    t = jnp.maximum(aw - qv, 0.0)
                    accs[h] = accs[h] + t * w2
            scores = []
            for h in range(H):
                sc = jnp.exp(jnp.minimum(jnp.maximum(accs[h] + b2v,
                                                     -5.0), 5.0))
                scores.append(sc)
                plsc.store_scatter(
                    msg, [rows, jnp.full((16,), D + h, jnp.int32)], sc)
            for d in range(DH):
                for h in range(H):
                    colv = jnp.full((16,), h * DH + d, jnp.int32)
                    vp = plsc.load_gather(msg, [rows, colv])
                    plsc.store_scatter(msg, [rows, colv], vp * scores[h])
            return 0

        lax.fori_loop(0, GPB, group, 0)
        pltpu.async_copy(msg, acc.at[idxb.at[4 + r]], sems[k], add=True)

    issue_idx(0)
    issue_idx(1)
    drain_idx(0)
    issue_gathers(0, 0)
    issue_idx(2)

    def run_batch(b, k):
        drain_gathers(k)
        drain_idx(b + 1)

        @pl.when(b >= 1)
        def _():
            wait_scatter(1 - k)

        issue_gathers(b + 1, 1 - k)

        @pl.when(b + 3 < NB)
        def _():
            issue_idx(b + 3)

        compute_batch(b, k)

    def pair(bp, _):
        b0 = bp * 2
        run_batch(b0, 0)
        run_batch(b0 + 1, 1)
        return 0

    lax.fori_loop(0, (NB - 1) // 2, pair, 0)
    for b in range((NB - 1) // 2 * 2, NB - 1):
        run_batch(b, b % 2)
    drain_gathers((NB - 1) % 2)
    wait_scatter(NB % 2)
    compute_batch(NB - 1, (NB - 1) % 2)
    wait_scatter((NB - 1) % 2)
    plsc.subcore_barrier()

    def dchunk(i, _):
        r0 = (sid + i * NS) * CH

        @pl.when(cid == 0)
        def _():
            pltpu.sync_copy(acc.at[pl.ds(r0, CH)], out0.at[pl.ds(r0, CH)])

        @pl.when(cid == 1)
        def _():
            pltpu.sync_copy(acc.at[pl.ds(r0, CH)], out1.at[pl.ds(r0, CH)])

        return 0

    lax.fori_loop(0, nchk_t, dchunk, 0)


_edge_call = functools.partial(
    pl.kernel,
    out_type=(
        jax.ShapeDtypeStruct((N2, ACCW), jnp.float32),
        jax.ShapeDtypeStruct((N2, ACCW), jnp.float32),
    ),
    mesh=plsc.VectorSubcoreMesh(core_axis_name="c", subcore_axis_name="s"),
    compiler_params=pltpu.CompilerParams(
        use_tc_tiling_on_sc=False, needs_layout_passes=False),
    scratch_types=[
        pltpu.VMEM((8, EB), jnp.int32),
        pltpu.VMEM((EB, D), jnp.float32),
        pltpu.VMEM((EB, D), jnp.float32),
        pltpu.VMEM((EB, D), jnp.float32),
        pltpu.VMEM((EB, D), jnp.float32),
        pltpu.VMEM((EB, ACCW), jnp.float32),
        pltpu.VMEM((EB, ACCW), jnp.float32),
        pltpu.VMEM((32,), jnp.float32),
        pltpu.VMEM_SHARED((N2, ACCW), jnp.float32),
        pltpu.SemaphoreType.DMA,
        pltpu.SemaphoreType.DMA,
        pltpu.SemaphoreType.DMA,
        pltpu.SemaphoreType.DMA,
        pltpu.SemaphoreType.DMA,
    ],
)(_edge_body)


def _s3_body(a0_ref, a1_ref, r8_ref, wo_ref, bo_ref, o_ref):
    wv = a0_ref[:, :D] + a1_ref[:, :D]
    z = a0_ref[:, D:D + H] + a1_ref[:, D:D + H]
    zr = (1.0 / z) @ r8_ref[...]
    o_ref[...] = (wv * zr) @ wo_ref[...] + bo_ref[...]


_s3_call = pl.pallas_call(
    _s3_body,
    grid=(N // _R1,),
    in_specs=[
        pl.BlockSpec((_R1, ACCW), lambda i: (i, 0)),
        pl.BlockSpec((_R1, ACCW), lambda i: (i, 0)),
        pl.BlockSpec((H, D), lambda i: (0, 0)),
        pl.BlockSpec((D, D), lambda i: (0, 0)),
        pl.BlockSpec((1, D), lambda i: (0, 0)),
    ],
    out_specs=pl.BlockSpec((_R1, D), lambda i: (i, 0)),
    out_shape=jax.ShapeDtypeStruct((N, D), jnp.float32),
)


def kernel(h, c, edge_index, Wq, Wk, Wv, Wp1, bp1, Wp2, bp2,
           We1, be1, We2, be2, Wo, bo):
    f32 = jnp.float32
    bd = jnp.kron(jnp.eye(H, dtype=f32), We1)
    wsrc = jnp.concatenate([Wk @ bd, Wv], axis=1)
    wp2src = jnp.concatenate([Wp2 @ bd, Wp2], axis=1)
    wp2src = jnp.pad(wp2src, ((0, 5), (0, 0)))
    bsrc = jnp.concatenate([bp2 @ bd, bp2]).reshape(1, 2 * D)
    wqq = Wq @ bd
    bqw = (-jnp.tile(be1, H)).reshape(1, D)
    wp1p = jnp.pad(Wp1, ((0, 5), (0, 5)))
    bp1p = jnp.pad(bp1, (0, 5)).reshape(1, 8)
    cpad = jnp.pad(c, ((0, 0), (0, 5)))

    awn, vpn, qwn = _s1_call(h, cpad, wp1p, bp1p, wsrc, wp2src, bsrc,
                             wqq, bqw)

    cons = jnp.concatenate([We2[:, 0] * 0.25,
                            jnp.full((16,), be2[0] * 0.25, f32)])
    srcp = jnp.pad(edge_index[0].reshape(NW, EPW),
                   ((0, 0), (0, LPW - EPW)))
    dstp = jnp.pad(edge_index[1].reshape(NW, EPW),
                   ((0, 0), (0, LPW - EPW)), constant_values=N)
    a0, a1 = _edge_call(srcp, dstp, awn, vpn, qwn, cons)

    r8 = jnp.kron(jnp.eye(H, dtype=f32), jnp.ones((1, DH), f32))
    return _s3_call(a0[:N], a1[:N], r8, Wo, bo.reshape(1, D))


# speedup vs baseline: 49.0306x; 2.4497x over previous
import functools

import jax
import jax.numpy as jnp
from jax import lax
from jax.experimental import pallas as pl
from jax.experimental.pallas import tpu as pltpu
from jax.experimental.pallas import tpu_sc as plsc

N = 10000
E = 320000
D = 128
H = 8
DH = 16

NC = 2
NS = 16
NW = NC * NS
EPW = E // NW
EB = 48
NB = -(-EPW // EB)
LPW = NB * EB
GPB = EB // 16
ACCW = 144
N2 = N + 8
CH = 24
NCHK = N2 // CH


def _s1_body(h_ref, c_ref, wp1_ref, bp1_ref, wsrc_ref, wp2_ref, bsrc_ref,
             wqq_ref, bqw_ref, oaw_ref, ovp_ref, oqw_ref):
    hb = h_ref[...]
    p2 = jnp.maximum(c_ref[...] @ wp1_ref[...] + bp1_ref[...], 0.0)
    big = hb @ wsrc_ref[...] + p2 @ wp2_ref[...] + bsrc_ref[...]
    oaw_ref[...] = big[:, :D]
    ovp_ref[...] = jnp.concatenate(
        [big[:, D:], jnp.zeros((big.shape[0], ACCW - D), jnp.float32)],
        axis=1)
    oqw_ref[...] = hb @ wqq_ref[...] + bqw_ref[...]


_R1 = 1000
_s1_call = pl.pallas_call(
    _s1_body,
    grid=(N // _R1,),
    in_specs=[
        pl.BlockSpec((_R1, D), lambda i: (i, 0)),
        pl.BlockSpec((_R1, 8), lambda i: (i, 0)),
        pl.BlockSpec((8, 8), lambda i: (0, 0)),
        pl.BlockSpec((1, 8), lambda i: (0, 0)),
        pl.BlockSpec((D, 2 * D), lambda i: (0, 0)),
        pl.BlockSpec((8, 2 * D), lambda i: (0, 0)),
        pl.BlockSpec((1, 2 * D), lambda i: (0, 0)),
        pl.BlockSpec((D, D), lambda i: (0, 0)),
        pl.BlockSpec((1, D), lambda i: (0, 0)),
    ],
    out_specs=[
        pl.BlockSpec((_R1, D), lambda i: (i, 0)),
        pl.BlockSpec((_R1, ACCW), lambda i: (i, 0)),
        pl.BlockSpec((_R1, D), lambda i: (i, 0)),
    ],
    out_shape=[
        jax.ShapeDtypeStruct((N, D), jnp.float32),
        jax.ShapeDtypeStruct((N, ACCW), jnp.float32),
        jax.ShapeDtypeStruct((N, D), jnp.float32),
    ],
)


def _edge_body(src_hbm, dst_hbm, aw_hbm, vp_hbm, qw_hbm, cons_hbm, out0, out1,
               idxb, rb0, rb1, qw0, qw1, msg0, msg1, cons, acc,
               semi, semg0, semg1, sems0, sems1):
    cid = lax.axis_index("c")
    sid = lax.axis_index("s")
    wid = cid * NS + sid
    rbs, qws, msgs = (rb0, rb1), (qw0, qw1), (msg0, msg1)
    semg, sems = (semg0, semg1), (sems0, sems1)

    pltpu.sync_copy(cons_hbm, cons)

    zv = jnp.zeros((16,), jnp.float32)

    def mrow(r, _):
        for c9 in range(ACCW // 16):
            msg0[r, pl.ds(c9 * 16, 16)] = zv
        return 0

    lax.fori_loop(0, CH, mrow, 0)

    base_chunks = NCHK // NS
    nchk_t = base_chunks + jnp.int32(sid < NCHK - base_chunks * NS)

    def zchunk(i, _):
        pltpu.sync_copy(msg0.at[pl.ds(0, CH)],
                        acc.at[pl.ds((sid + i * NS) * CH, CH)])
        return 0

    lax.fori_loop(0, nchk_t, zchunk, 0)
    plsc.subcore_barrier()

    iota16 = lax.broadcasted_iota(jnp.int32, (16,), 0)

    w2v = cons[pl.ds(0, 16)]
    b2v = cons[pl.ds(16, 16)]

    def issue_idx(b):
        r = b % 4
        pltpu.async_copy(src_hbm.at[wid, pl.ds(b * EB, EB)],
                         idxb.at[r], semi)
        pltpu.async_copy(dst_hbm.at[wid, pl.ds(b * EB, EB)],
                         idxb.at[4 + r], semi)

    def drain_idx(b):
        r = b % 4
        pltpu.make_async_copy(
            src_hbm.at[0, pl.ds(0, EB)], idxb.at[r], semi).wait()
        pltpu.make_async_copy(
            src_hbm.at[0, pl.ds(0, EB)], idxb.at[4 + r], semi).wait()

    def issue_gathers(b, k):
        r = b % 4
        pltpu.async_copy(aw_hbm.at[idxb.at[r]], rbs[k], semg[k])
        pltpu.async_copy(qw_hbm.at[idxb.at[4 + r]], qws[k], semg[k])
        pltpu.async_copy(vp_hbm.at[idxb.at[r]], msgs[k], semg[k])

    def drain_gathers(k):
        pltpu.make_async_copy(aw_hbm.at[idxb.at[0]], rbs[k],
                              semg[k]).wait()
        pltpu.make_async_copy(qw_hbm.at[idxb.at[4]], qws[k],
                              semg[k]).wait()
        pltpu.make_async_copy(vp_hbm.at[idxb.at[0]], msgs[k],
                              semg[k]).wait()

    def wait_scatter(k):
        pltpu.make_async_copy(msgs[k], acc.at[idxb.at[4]],
                              sems[k]).wait()

    def compute_batch(b, k):
        r = b % 4
        rb, qwr, msg = rbs[k], qws[k], msgs[k]

        def edge(e, _):
            svec = jnp.zeros((16,), jnp.float32)
            for h in range(H):
                aw = rb[e, pl.ds(h * DH, DH)]
                qv = qwr[e, pl.ds(h * DH, DH)]
                t = jnp.maximum(aw - qv, 0.0)
                s = jnp.sum(t * w2v)
                svec = jnp.where(iota16 == h, s, svec)
            svec = jnp.exp(jnp.minimum(jnp.maximum(svec + b2v, -5.0), 5.0))
            svec = jnp.where(iota16 < H, svec, 0.0)
            msg[e, pl.ds(D, 16)] = svec
            for h in range(H):
                sc = svec[h]
                vp = msg[e, pl.ds(h * DH, DH)]
                msg[e, pl.ds(h * DH, DH)] = vp * sc
            return 0

        lax.fori_loop(0, EB, edge, 0)
        pltpu.async_copy(msg, acc.at[idxb.at[4 + r]], sems[k], add=True)

    issue_idx(0)
    issue_idx(1)
    drain_idx(0)
    issue_gathers(0, 0)
    issue_idx(2)

    def run_batch(b, k):
        drain_gathers(k)
        drain_idx(b + 1)

        @pl.when(b >= 1)
        def _():
            wait_scatter(1 - k)

        issue_gathers(b + 1, 1 - k)

        @pl.when(b + 3 < NB)
        def _():
            issue_idx(b + 3)

        compute_batch(b, k)

    def pair(bp, _):
        b0 = bp * 2
        run_batch(b0, 0)
        run_batch(b0 + 1, 1)
        return 0

    lax.fori_loop(0, (NB - 1) // 2, pair, 0)
    for b in range((NB - 1) // 2 * 2, NB - 1):
        run_batch(b, b % 2)
    drain_gathers((NB - 1) % 2)
    wait_scatter(NB % 2)
    compute_batch(NB - 1, (NB - 1) % 2)
    wait_scatter((NB - 1) % 2)
    plsc.subcore_barrier()

    def dchunk(i, _):
        r0 = (sid + i * NS) * CH

        @pl.when(cid == 0)
        def _():
            pltpu.sync_copy(acc.at[pl.ds(r0, CH)], out0.at[pl.ds(r0, CH)])

        @pl.when(cid == 1)
        def _():
            pltpu.sync_copy(acc.at[pl.ds(r0, CH)], out1.at[pl.ds(r0, CH)])

        return 0

    lax.fori_loop(0, nchk_t, dchunk, 0)


_edge_call = functools.partial(
    pl.kernel,
    out_type=(
        jax.ShapeDtypeStruct((N2, ACCW), jnp.float32),
        jax.ShapeDtypeStruct((N2, ACCW), jnp.float32),
    ),
    mesh=plsc.VectorSubcoreMesh(core_axis_name="c", subcore_axis_name="s"),
    compiler_params=pltpu.CompilerParams(
        use_tc_tiling_on_sc=False, needs_layout_passes=False),
    scratch_types=[
        pltpu.VMEM((8, EB), jnp.int32),
        pltpu.VMEM((EB, D), jnp.float32),
        pltpu.VMEM((EB, D), jnp.float32),
        pltpu.VMEM((EB, D), jnp.float32),
        pltpu.VMEM((EB, D), jnp.float32),
        pltpu.VMEM((EB, ACCW), jnp.float32),
        pltpu.VMEM((EB, ACCW), jnp.float32),
        pltpu.VMEM((32,), jnp.float32),
        pltpu.VMEM_SHARED((N2, ACCW), jnp.float32),
        pltpu.SemaphoreType.DMA,
        pltpu.SemaphoreType.DMA,
        pltpu.SemaphoreType.DMA,
        pltpu.SemaphoreType.DMA,
        pltpu.SemaphoreType.DMA,
    ],
)(_edge_body)


def _s3_body(a0_ref, a1_ref, r8_ref, wo_ref, bo_ref, o_ref):
    wv = a0_ref[:, :D] + a1_ref[:, :D]
    z = a0_ref[:, D:D + H] + a1_ref[:, D:D + H]
    zr = (1.0 / z) @ r8_ref[...]
    o_ref[...] = (wv * zr) @ wo_ref[...] + bo_ref[...]


_s3_call = pl.pallas_call(
    _s3_body,
    grid=(N // _R1,),
    in_specs=[
        pl.BlockSpec((_R1, ACCW), lambda i: (i, 0)),
        pl.BlockSpec((_R1, ACCW), lambda i: (i, 0)),
        pl.BlockSpec((H, D), lambda i: (0, 0)),
        pl.BlockSpec((D, D), lambda i: (0, 0)),
        pl.BlockSpec((1, D), lambda i: (0, 0)),
    ],
    out_specs=pl.BlockSpec((_R1, D), lambda i: (i, 0)),
    out_shape=jax.ShapeDtypeStruct((N, D), jnp.float32),
)


def kernel(h, c, edge_index, Wq, Wk, Wv, Wp1, bp1, Wp2, bp2,
           We1, be1, We2, be2, Wo, bo):
    f32 = jnp.float32
    bd = jnp.kron(jnp.eye(H, dtype=f32), We1)
    wsrc = jnp.concatenate([Wk @ bd, Wv], axis=1)
    wp2src = jnp.concatenate([Wp2 @ bd, Wp2], axis=1)
    wp2src = jnp.pad(wp2src, ((0, 5), (0, 0)))
    bsrc = jnp.concatenate([bp2 @ bd, bp2]).reshape(1, 2 * D)
    wqq = Wq @ bd
    bqw = (-jnp.tile(be1, H)).reshape(1, D)
    wp1p = jnp.pad(Wp1, ((0, 5), (0, 5)))
    bp1p = jnp.pad(bp1, (0, 5)).reshape(1, 8)
    cpad = jnp.pad(c, ((0, 0), (0, 5)))

    awn, vpn, qwn = _s1_call(h, cpad, wp1p, bp1p, wsrc, wp2src, bsrc,
                             wqq, bqw)

    cons = jnp.concatenate([We2[:, 0] * 0.25,
                            jnp.full((16,), be2[0] * 0.25, f32)])
    srcp = jnp.pad(edge_index[0].reshape(NW, EPW),
                   ((0, 0), (0, LPW - EPW)))
    dstp = jnp.pad(edge_index[1].reshape(NW, EPW),
                   ((0, 0), (0, LPW - EPW)), constant_values=N)
    a0, a1 = _edge_call(srcp, dstp, awn, vpn, qwn, cons)

    r8 = jnp.kron(jnp.eye(H, dtype=f32), jnp.ones((1, DH), f32))
    return _s3_call(a0[:N], a1[:N], r8, Wo, bo.reshape(1, D))


# speedup vs baseline: 49.2687x; 1.0049x over previous
import functools

import jax
import jax.numpy as jnp
from jax import lax
from jax.experimental import pallas as pl
from jax.experimental.pallas import tpu as pltpu
from jax.experimental.pallas import tpu_sc as plsc

N = 10000
E = 320000
D = 128
H = 8
DH = 16

NC = 2
NS = 16
NW = NC * NS
EPW = E // NW
EB = 48
NB = -(-EPW // EB)
LPW = NB * EB
GPB = EB // 16
ACCW = 144
N2 = N + 8
CH = 24
NCHK = N2 // CH


def _s1_body(h_ref, c_ref, wp1_ref, bp1_ref, wsrc_ref, wp2_ref, bsrc_ref,
             wqq_ref, bqw_ref, oaw_ref, ovp_ref, oqw_ref):
    hb = h_ref[...]
    p2 = jnp.maximum(c_ref[...] @ wp1_ref[...] + bp1_ref[...], 0.0)
    big = hb @ wsrc_ref[...] + p2 @ wp2_ref[...] + bsrc_ref[...]
    oaw_ref[...] = big[:, :D]
    ovp_ref[...] = jnp.concatenate(
        [big[:, D:], jnp.zeros((big.shape[0], ACCW - D), jnp.float32)],
        axis=1)
    oqw_ref[...] = hb @ wqq_ref[...] + bqw_ref[...]


_R1 = 1000
_s1_call = pl.pallas_call(
    _s1_body,
    grid=(N // _R1,),
    in_specs=[
        pl.BlockSpec((_R1, D), lambda i: (i, 0)),
        pl.BlockSpec((_R1, 8), lambda i: (i, 0)),
        pl.BlockSpec((8, 8), lambda i: (0, 0)),
        pl.BlockSpec((1, 8), lambda i: (0, 0)),
        pl.BlockSpec((D, 2 * D), lambda i: (0, 0)),
        pl.BlockSpec((8, 2 * D), lambda i: (0, 0)),
        pl.BlockSpec((1, 2 * D), lambda i: (0, 0)),
        pl.BlockSpec((D, D), lambda i: (0, 0)),
        pl.BlockSpec((1, D), lambda i: (0, 0)),
    ],
    out_specs=[
        pl.BlockSpec((_R1, D), lambda i: (i, 0)),
        pl.BlockSpec((_R1, ACCW), lambda i: (i, 0)),
        pl.BlockSpec((_R1, D), lambda i: (i, 0)),
    ],
    out_shape=[
        jax.ShapeDtypeStruct((N, D), jnp.float32),
        jax.ShapeDtypeStruct((N, ACCW), jnp.float32),
        jax.ShapeDtypeStruct((N, D), jnp.float32),
    ],
)


def _edge_body(src_hbm, dst_hbm, aw_hbm, vp_hbm, qw_hbm, cons_hbm, out0, out1,
               idxb, rb0, rb1, qw0, qw1, msg0, msg1, cons, acc,
               semi, semg0, semg1, sems0, sems1):
    cid = lax.axis_index("c")
    sid = lax.axis_index("s")
    wid = cid * NS + sid
    rbs, qws, msgs = (rb0, rb1), (qw0, qw1), (msg0, msg1)
    semg, sems = (semg0, semg1), (sems0, sems1)

    pltpu.sync_copy(cons_hbm, cons)

    zv = jnp.zeros((16,), jnp.float32)

    def mrow(r, _):
        for c9 in range(ACCW // 16):
            msg0[r, pl.ds(c9 * 16, 16)] = zv
        return 0

    lax.fori_loop(0, CH, mrow, 0)

    base_chunks = NCHK // NS
    nchk_t = base_chunks + jnp.int32(sid < NCHK - base_chunks * NS)

    def zchunk(i, _):
        pltpu.sync_copy(msg0.at[pl.ds(0, CH)],
                        acc.at[pl.ds((sid + i * NS) * CH, CH)])
        return 0

    lax.fori_loop(0, nchk_t, zchunk, 0)
    plsc.subcore_barrier()

    iota16 = lax.broadcasted_iota(jnp.int32, (16,), 0)

    w2v = cons[pl.ds(0, 16)]
    b2v = cons[pl.ds(16, 16)]

    def issue_idx(b):
        r = b % 4
        pltpu.async_copy(src_hbm.at[wid, pl.ds(b * EB, EB)],
                         idxb.at[r], semi)
        pltpu.async_copy(dst_hbm.at[wid, pl.ds(b * EB, EB)],
                         idxb.at[4 + r], semi)

    def drain_idx(b):
        r = b % 4
        pltpu.make_async_copy(
            src_hbm.at[0, pl.ds(0, EB)], idxb.at[r], semi).wait()
        pltpu.make_async_copy(
            src_hbm.at[0, pl.ds(0, EB)], idxb.at[4 + r], semi).wait()

    def issue_gathers(b, k):
        r = b % 4
        pltpu.async_copy(aw_hbm.at[idxb.at[r]], rbs[k], semg[k])
        pltpu.async_copy(qw_hbm.at[idxb.at[4 + r]], qws[k], semg[k])
        pltpu.async_copy(vp_hbm.at[idxb.at[r]], msgs[k], semg[k])

    def drain_gathers(k):
        pltpu.make_async_copy(aw_hbm.at[idxb.at[0]], rbs[k],
                              semg[k]).wait()
        pltpu.make_async_copy(qw_hbm.at[idxb.at[4]], qws[k],
                              semg[k]).wait()
        pltpu.make_async_copy(vp_hbm.at[idxb.at[0]], msgs[k],
                              semg[k]).wait()

    def wait_scatter(k):
        pltpu.make_async_copy(msgs[k], acc.at[idxb.at[4]],
                              sems[k]).wait()

    def compute_batch(b, k):
        r = b % 4
        rb, qwr, msg = rbs[k], qws[k], msgs[k]

        def edge(e2, _):
            for j in range(2):
                e = e2 * 2 + j
                svec = jnp.zeros((16,), jnp.float32)
                for h in range(H):
                    aw = rb[e, pl.ds(h * DH, DH)]
                    qv = qwr[e, pl.ds(h * DH, DH)]
                    t = jnp.maximum(aw - qv, 0.0)
                    s = jnp.sum(t * w2v)
                    svec = jnp.where(iota16 == h, s, svec)
                svec = jnp.exp(
                    jnp.minimum(jnp.maximum(svec + b2v, -5.0), 5.0))
                svec = jnp.where(iota16 < H, svec, 0.0)
                msg[e, pl.ds(D, 16)] = svec
                for h in range(H):
                    sc = svec[h]
                    vp = msg[e, pl.ds(h * DH, DH)]
                    msg[e, pl.ds(h * DH, DH)] = vp * sc
            return 0

        lax.fori_loop(0, EB // 2, edge, 0)
        pltpu.async_copy(msg, acc.at[idxb.at[4 + r]], sems[k], add=True)

    issue_idx(0)
    issue_idx(1)
    drain_idx(0)
    issue_gathers(0, 0)
    issue_idx(2)

    def run_batch(b, k):
        drain_gathers(k)
        drain_idx(b + 1)

        @pl.when(b >= 1)
        def _():
            wait_scatter(1 - k)

        issue_gathers(b + 1, 1 - k)

        @pl.when(b + 3 < NB)
        def _():
            issue_idx(b + 3)

        compute_batch(b, k)

    def pair(bp, _):
        b0 = bp * 2
        run_batch(b0, 0)
        run_batch(b0 + 1, 1)
        return 0

    lax.fori_loop(0, (NB - 1) // 2, pair, 0)
    for b in range((NB - 1) // 2 * 2, NB - 1):
        run_batch(b, b % 2)
    drain_gathers((NB - 1) % 2)
    wait_scatter(NB % 2)
    compute_batch(NB - 1, (NB - 1) % 2)
    wait_scatter((NB - 1) % 2)
    plsc.subcore_barrier()

    def dchunk(i, _):
        r0 = (sid + i * NS) * CH

        @pl.when(cid == 0)
        def _():
            pltpu.sync_copy(acc.at[pl.ds(r0, CH)], out0.at[pl.ds(r0, CH)])

        @pl.when(cid == 1)
        def _():
            pltpu.sync_copy(acc.at[pl.ds(r0, CH)], out1.at[pl.ds(r0, CH)])

        return 0

    lax.fori_loop(0, nchk_t, dchunk, 0)


_edge_call = functools.partial(
    pl.kernel,
    out_type=(
        jax.ShapeDtypeStruct((N2, ACCW), jnp.float32),
        jax.ShapeDtypeStruct((N2, ACCW), jnp.float32),
    ),
    mesh=plsc.VectorSubcoreMesh(core_axis_name="c", subcore_axis_name="s"),
    compiler_params=pltpu.CompilerParams(
        use_tc_tiling_on_sc=False, needs_layout_passes=False),
    scratch_types=[
        pltpu.VMEM((8, EB), jnp.int32),
        pltpu.VMEM((EB, D), jnp.float32),
        pltpu.VMEM((EB, D), jnp.float32),
        pltpu.VMEM((EB, D), jnp.float32),
        pltpu.VMEM((EB, D), jnp.float32),
        pltpu.VMEM((EB, ACCW), jnp.float32),
        pltpu.VMEM((EB, ACCW), jnp.float32),
        pltpu.VMEM((32,), jnp.float32),
        pltpu.VMEM_SHARED((N2, ACCW), jnp.float32),
        pltpu.SemaphoreType.DMA,
        pltpu.SemaphoreType.DMA,
        pltpu.SemaphoreType.DMA,
        pltpu.SemaphoreType.DMA,
        pltpu.SemaphoreType.DMA,
    ],
)(_edge_body)


def _s3_body(a0_ref, a1_ref, r8_ref, wo_ref, bo_ref, o_ref):
    wv = a0_ref[:, :D] + a1_ref[:, :D]
    z = a0_ref[:, D:D + H] + a1_ref[:, D:D + H]
    zr = (1.0 / z) @ r8_ref[...]
    o_ref[...] = (wv * zr) @ wo_ref[...] + bo_ref[...]


_s3_call = pl.pallas_call(
    _s3_body,
    grid=(N // _R1,),
    in_specs=[
        pl.BlockSpec((_R1, ACCW), lambda i: (i, 0)),
        pl.BlockSpec((_R1, ACCW), lambda i: (i, 0)),
        pl.BlockSpec((H, D), lambda i: (0, 0)),
        pl.BlockSpec((D, D), lambda i: (0, 0)),
        pl.BlockSpec((1, D), lambda i: (0, 0)),
    ],
    out_specs=pl.BlockSpec((_R1, D), lambda i: (i, 0)),
    out_shape=jax.ShapeDtypeStruct((N, D), jnp.float32),
)


def kernel(h, c, edge_index, Wq, Wk, Wv, Wp1, bp1, Wp2, bp2,
           We1, be1, We2, be2, Wo, bo):
    f32 = jnp.float32
    bd = jnp.kron(jnp.eye(H, dtype=f32), We1)
    wsrc = jnp.concatenate([Wk @ bd, Wv], axis=1)
    wp2src = jnp.concatenate([Wp2 @ bd, Wp2], axis=1)
    wp2src = jnp.pad(wp2src, ((0, 5), (0, 0)))
    bsrc = jnp.concatenate([bp2 @ bd, bp2]).reshape(1, 2 * D)
    wqq = Wq @ bd
    bqw = (-jnp.tile(be1, H)).reshape(1, D)
    wp1p = jnp.pad(Wp1, ((0, 5), (0, 5)))
    bp1p = jnp.pad(bp1, (0, 5)).reshape(1, 8)
    cpad = jnp.pad(c, ((0, 0), (0, 5)))

    awn, vpn, qwn = _s1_call(h, cpad, wp1p, bp1p, wsrc, wp2src, bsrc,
                             wqq, bqw)

    cons = jnp.concatenate([We2[:, 0] * 0.25,
                            jnp.full((16,), be2[0] * 0.25, f32)])
    srcp = jnp.pad(edge_index[0].reshape(NW, EPW),
                   ((0, 0), (0, LPW - EPW)))
    dstp = jnp.pad(edge_index[1].reshape(NW, EPW),
                   ((0, 0), (0, LPW - EPW)), constant_values=N)
    a0, a1 = _edge_call(srcp, dstp, awn, vpn, qwn, cons)

    r8 = jnp.kron(jnp.eye(H, dtype=f32), jnp.ones((1, DH), f32))
    return _s3_call(a0[:N], a1[:N], r8, Wo, bo.reshape(1, D))
